# Initial kernel scaffold; baseline (speedup 1.0000x reference)
import jax
import jax.numpy as jnp
from jax.experimental import pallas as pl


def kernel(bbox, anchor, img_h, img_w):
    raise NotImplementedError("write your pallas kernel here")


# speedup vs baseline: 4.9066x; 4.9066x over previous
import jax
import jax.numpy as jnp
from jax.experimental import pallas as pl
from jax.experimental.pallas import tpu as pltpu

_N_SAMPLE = 256
_POS_IOU_THRESH = 0.7
_NEG_IOU_THRESH = 0.3
_N_POS = 128

_R = 157
_C = 128
_NP = _R * _C
_G = 32


def _body(hw_ref, bbox_ref, a_ref, loc_ref, lab_ref):
    H = hw_ref[0, 0]
    W = hw_ref[0, 1]
    ay1 = a_ref[0]
    ax1 = a_ref[1]
    ay2 = a_ref[2]
    ax2 = a_ref[3]
    inside = (ay1 >= 0.0) & (ax1 >= 0.0) & (ay2 <= H) & (ax2 <= W)
    area_a = (ay2 - ay1) * (ax2 - ax1)

    max_ious = jnp.full((_R, _C), -1.0, jnp.float32)
    argmax = jnp.zeros((_R, _C), jnp.int32)
    gt_mask = jnp.zeros((_R, _C), jnp.bool_)
    my1 = jnp.full((_R, _C), bbox_ref[0, 0], jnp.float32)
    mx1 = jnp.full((_R, _C), bbox_ref[0, 1], jnp.float32)
    my2 = jnp.full((_R, _C), bbox_ref[0, 2], jnp.float32)
    mx2 = jnp.full((_R, _C), bbox_ref[0, 3], jnp.float32)

    for g in range(_G):
        by1 = bbox_ref[g, 0]
        bx1 = bbox_ref[g, 1]
        by2 = bbox_ref[g, 2]
        bx2 = bbox_ref[g, 3]
        tly = jnp.maximum(ay1, by1)
        tlx = jnp.maximum(ax1, bx1)
        bry = jnp.minimum(ay2, by2)
        brx = jnp.minimum(ax2, bx2)
        valid = (tly < bry) & (tlx < brx)
        area_i = jnp.where(valid, (bry - tly) * (brx - tlx), 0.0)
        area_b = (by2 - by1) * (bx2 - bx1)
        iou = area_i / (area_a + area_b - area_i)
        iou_m = jnp.where(inside, iou, -1.0)
        upd = iou_m > max_ious
        argmax = jnp.where(upd, g, argmax)
        max_ious = jnp.maximum(max_ious, iou_m)
        my1 = jnp.where(upd, by1, my1)
        mx1 = jnp.where(upd, bx1, mx1)
        my2 = jnp.where(upd, by2, my2)
        mx2 = jnp.where(upd, bx2, mx2)
        gmax = jnp.max(iou_m)
        gt_mask = gt_mask | (iou_m == gmax)
    del argmax

    neg = inside & (max_ious < _NEG_IOU_THRESH) & (max_ious >= 0.0)
    pos = (gt_mask & inside) | (inside & (max_ious >= _POS_IOU_THRESH))
    label = jnp.where(pos, 1, jnp.where(neg, 0, -1)).astype(jnp.int32)

    ki = jax.lax.broadcasted_iota(jnp.int32, (_C, _C), 0)
    ci = jax.lax.broadcasted_iota(jnp.int32, (_C, _C), 1)
    T = (ki <= ci).astype(jnp.float32)
    ri = jax.lax.broadcasted_iota(jnp.int32, (_R, _R), 0)
    si = jax.lax.broadcasted_iota(jnp.int32, (_R, _R), 1)
    M = (si < ri).astype(jnp.float32)

    def cumsum(x):
        p = jax.lax.dot(x, T, preferred_element_type=jnp.float32)
        rowtot = jnp.broadcast_to(p[:, _C - 1:_C], (_R, _C))
        offs = jax.lax.dot(M, rowtot, preferred_element_type=jnp.float32)
        return p + offs

    posf = (label == 1).astype(jnp.float32)
    pos_cum = cumsum(posf)
    total_pos = jnp.sum(posf)
    label = jnp.where((label == 1) & (pos_cum > float(_N_POS)), -1, label)
    n_neg = float(_N_SAMPLE) - jnp.minimum(total_pos, float(_N_POS))
    negf = (label == 0).astype(jnp.float32)
    neg_cum = cumsum(negf)
    label = jnp.where((label == 0) & (neg_cum > n_neg), -1, label)

    eps = jnp.float32(jnp.finfo(jnp.float32).eps)
    h = ay2 - ay1
    w = ax2 - ax1
    cy = ay1 + 0.5 * h
    cx = ax1 + 0.5 * w
    bh = my2 - my1
    bw = mx2 - mx1
    bcy = my1 + 0.5 * bh
    bcx = mx1 + 0.5 * bw
    h = jnp.maximum(h, eps)
    w = jnp.maximum(w, eps)
    dy = (bcy - cy) / h
    dx = (bcx - cx) / w
    dh = jnp.log(bh / h)
    dw = jnp.log(bw / w)

    zero = jnp.zeros((_R, _C), jnp.float32)
    loc_ref[0] = jnp.where(inside, dy, zero)
    loc_ref[1] = jnp.where(inside, dx, zero)
    loc_ref[2] = jnp.where(inside, dh, zero)
    loc_ref[3] = jnp.where(inside, dw, zero)
    lab_ref[...] = label


def kernel(bbox, anchor, img_h, img_w):
    N = anchor.shape[0]
    pad = jnp.full((_NP - N, 4), -1.0, jnp.float32)
    aT = jnp.concatenate([anchor.astype(jnp.float32), pad], axis=0).T
    aT = aT.reshape(4, _R, _C)
    hw = jnp.stack([img_h, img_w]).astype(jnp.float32).reshape(1, 2)

    loc4, lab = pl.pallas_call(
        _body,
        out_shape=[
            jax.ShapeDtypeStruct((4, _R, _C), jnp.float32),
            jax.ShapeDtypeStruct((_R, _C), jnp.int32),
        ],
        in_specs=[
            pl.BlockSpec(memory_space=pltpu.SMEM),
            pl.BlockSpec(memory_space=pltpu.SMEM),
            pl.BlockSpec(memory_space=pltpu.VMEM),
        ],
        out_specs=[
            pl.BlockSpec(memory_space=pltpu.VMEM),
            pl.BlockSpec(memory_space=pltpu.VMEM),
        ],
    )(hw, bbox.astype(jnp.float32), aT)

    loc = loc4.reshape(4, _NP).T[:N]
    label = lab.reshape(_NP)[:N]
    return loc, label
